# baseline (speedup 1.0000x reference)
import functools

import jax
import jax.numpy as jnp
from jax import lax
from jax.experimental import pallas as pl
from jax.experimental.pallas import tpu as pltpu
from jax.experimental.pallas import tpu_sc as plsc

NC = 2
NS = 16
NW = NC * NS
CH = 128

_f32 = jnp.float32


def _degrees(src3, dst3, n_pad):
    nw, nchunk, ch = src3.shape
    mesh = plsc.VectorSubcoreMesh(core_axis_name="c", subcore_axis_name="s")

    @functools.partial(
        pl.kernel,
        out_type=(
            jax.ShapeDtypeStruct((nw, n_pad), _f32),
            jax.ShapeDtypeStruct((nw, n_pad), _f32),
        ),
        mesh=mesh,
        compiler_params=pltpu.CompilerParams(
            needs_layout_passes=False, use_tc_tiling_on_sc=False),
        scratch_types=[
            pltpu.VMEM((nchunk, ch), jnp.int32),
            pltpu.VMEM((nchunk, ch), jnp.int32),
            pltpu.VMEM((n_pad,), _f32),
            pltpu.VMEM((n_pad,), _f32),
        ],
    )
    def deg_kernel(src_hbm, dst_hbm, out_s, out_d, src_v, dst_v, acc_s, acc_d):
        c = lax.axis_index("c")
        s = lax.axis_index("s")
        wid = c * NS + s
        pltpu.sync_copy(src_hbm.at[wid], src_v)
        pltpu.sync_copy(dst_hbm.at[wid], dst_v)
        zeros16 = jnp.zeros((16,), _f32)

        def zbody(i, carry):
            acc_s[pl.ds(i * 16, 16)] = zeros16
            acc_d[pl.ds(i * 16, 16)] = zeros16
            return carry

        lax.fori_loop(0, n_pad // 16, zbody, 0)
        ones16 = jnp.ones((16,), _f32)

        def cbody(i, carry):
            for j in range(ch // 16):
                si = src_v[i, pl.ds(j * 16, 16)]
                plsc.addupdate_scatter(acc_s, [si], ones16)
                di = dst_v[i, pl.ds(j * 16, 16)]
                plsc.addupdate_scatter(acc_d, [di], ones16)
            return carry

        lax.fori_loop(0, nchunk, cbody, 0)
        pltpu.sync_copy(acc_s, out_s.at[wid])
        pltpu.sync_copy(acc_d, out_d.at[wid])

    return deg_kernel(src3, dst3)


def _aggregate(t_pad, src3, dst3, zeros_blk, n):
    n_pad, d = t_pad.shape
    nw, nchunk, ch = src3.shape
    rps = n_pad // NS
    full_sub = n // rps
    rem = n - full_sub * rps
    mesh = plsc.VectorSubcoreMesh(core_axis_name="c", subcore_axis_name="s")

    @functools.partial(
        pl.kernel,
        out_type=jax.ShapeDtypeStruct((NC, n, d), _f32),
        mesh=mesh,
        compiler_params=pltpu.CompilerParams(
            needs_layout_passes=False, use_tc_tiling_on_sc=False),
        scratch_types=[
            pltpu.VMEM((nchunk // 2, ch), jnp.int32),
            pltpu.VMEM((nchunk // 2, ch), jnp.int32),
            pltpu.VMEM((ch, d), _f32),
            pltpu.VMEM((ch, d), _f32),
            pltpu.VMEM_SHARED((n_pad, d), _f32),
            pltpu.SemaphoreType.DMA,
            pltpu.SemaphoreType.DMA,
        ],
    )
    def agg_kernel(t_hbm, src_hbm, dst_hbm, z_hbm, out_hbm,
                   src_v, dst_v, rows0, rows1, acc_sh, sem0, sem1):
        c = lax.axis_index("c")
        s = lax.axis_index("s")
        wid = c * NS + s
        nhalf = nchunk // 2
        npairs = nhalf // 2

        def run_half(h):
            pltpu.sync_copy(src_hbm.at[wid, pl.ds(h * nhalf, nhalf)], src_v)
            pltpu.sync_copy(dst_hbm.at[wid, pl.ds(h * nhalf, nhalf)], dst_v)
            pltpu.async_copy(t_hbm.at[src_v.at[0]], rows0, sem0)
            pltpu.async_copy(t_hbm.at[src_v.at[1]], rows1, sem1)
            if h == 0:
                pltpu.sync_copy(z_hbm, acc_sh.at[pl.ds(s * rps, rps)])
                plsc.subcore_barrier()

            def body(p, carry):
                i0 = p * 2
                pltpu.make_async_copy(
                    t_hbm.at[src_v.at[i0]], rows0, sem0).wait()
                pltpu.sync_copy(rows0, acc_sh.at[dst_v.at[i0]], add=True)

                @pl.when(p < npairs - 1)
                def _():
                    pltpu.async_copy(t_hbm.at[src_v.at[i0 + 2]], rows0, sem0)

                pltpu.make_async_copy(
                    t_hbm.at[src_v.at[i0 + 1]], rows1, sem1).wait()
                pltpu.sync_copy(rows1, acc_sh.at[dst_v.at[i0 + 1]], add=True)

                @pl.when(p < npairs - 1)
                def _():
                    pltpu.async_copy(t_hbm.at[src_v.at[i0 + 3]], rows1, sem1)

                return carry

            lax.fori_loop(0, npairs, body, 0)

        run_half(0)
        run_half(1)
        plsc.subcore_barrier()
        row0 = s * rps

        @pl.when(s < full_sub)
        def _():
            pltpu.sync_copy(acc_sh.at[pl.ds(row0, rps)],
                            out_hbm.at[c, pl.ds(row0, rps)])

        if rem > 0:
            @pl.when(s == full_sub)
            def _():
                pltpu.sync_copy(acc_sh.at[pl.ds(full_sub * rps, rem)],
                                out_hbm.at[c, pl.ds(full_sub * rps, rem)])

    return agg_kernel(t_pad, src3, dst3, zeros_blk)


def _row_block(n):
    for b in (2000, 1000, 500, 250, 200, 100, 50, 25, 8):
        if n % b == 0:
            return b
    return n


def _norms(deg_s, deg_d, n):
    nw = deg_s.shape[0]

    def body(ds_ref, dd_ref, ns_ref, nd_ref):
        ones = jnp.ones((nw, 1), _f32)
        dn = (((0,), (0,)), ((), ()))
        ssum = lax.dot_general(ds_ref[...], ones, dn,
                               preferred_element_type=_f32)
        dsum = lax.dot_general(dd_ref[...], ones, dn,
                               preferred_element_type=_f32)
        ns_ref[...] = lax.rsqrt(jnp.maximum(ssum, 1.0))
        nd_ref[...] = lax.rsqrt(jnp.maximum(dsum, 1.0))

    return pl.pallas_call(
        body,
        out_shape=[
            jax.ShapeDtypeStruct((n, 1), _f32),
            jax.ShapeDtypeStruct((n, 1), _f32),
        ],
    )(deg_s, deg_d)


def _scale_matmul(x, ns, w, n):
    rb = _row_block(n)
    d_in = x.shape[1]
    d_out = w.shape[1]

    def body(x_ref, ns_ref, w_ref, o_ref):
        o_ref[...] = jnp.dot(x_ref[...] * ns_ref[...], w_ref[...],
                             preferred_element_type=_f32)

    return pl.pallas_call(
        body,
        grid=(n // rb,),
        in_specs=[
            pl.BlockSpec((rb, d_in), lambda i: (i, 0)),
            pl.BlockSpec((rb, 1), lambda i: (i, 0)),
            pl.BlockSpec((d_in, d_out), lambda i: (0, 0)),
        ],
        out_specs=pl.BlockSpec((rb, d_out), lambda i: (i, 0)),
        out_shape=jax.ShapeDtypeStruct((n, d_out), _f32),
    )(x, ns, w)


def _mid(p, nd, b1, ns, w2, n):
    rb = _row_block(n)
    nc, _, hid = p.shape
    d_out = w2.shape[1]

    def body(p_ref, nd_ref, b_ref, ns_ref, w_ref, o_ref):
        h = (p_ref[0] + p_ref[1]) * nd_ref[...] + b_ref[...]
        h = jnp.maximum(h, 0.0)
        o_ref[...] = jnp.dot(h * ns_ref[...], w_ref[...],
                             preferred_element_type=_f32)

    return pl.pallas_call(
        body,
        grid=(n // rb,),
        in_specs=[
            pl.BlockSpec((nc, rb, hid), lambda i: (0, i, 0)),
            pl.BlockSpec((rb, 1), lambda i: (i, 0)),
            pl.BlockSpec((1, hid), lambda i: (0, 0)),
            pl.BlockSpec((rb, 1), lambda i: (i, 0)),
            pl.BlockSpec((hid, d_out), lambda i: (0, 0)),
        ],
        out_specs=pl.BlockSpec((rb, d_out), lambda i: (i, 0)),
        out_shape=jax.ShapeDtypeStruct((n, d_out), _f32),
    )(p, nd, b1, ns, w2)


def _final(p, nd, b2, n):
    rb = _row_block(n)
    nc, _, d_out = p.shape

    def body(p_ref, nd_ref, b_ref, o_ref):
        o_ref[...] = (p_ref[0] + p_ref[1]) * nd_ref[...] + b_ref[...]

    return pl.pallas_call(
        body,
        grid=(n // rb,),
        in_specs=[
            pl.BlockSpec((nc, rb, d_out), lambda i: (0, i, 0)),
            pl.BlockSpec((rb, 1), lambda i: (i, 0)),
            pl.BlockSpec((1, d_out), lambda i: (0, 0)),
        ],
        out_specs=pl.BlockSpec((rb, d_out), lambda i: (i, 0)),
        out_shape=jax.ShapeDtypeStruct((n, d_out), _f32),
    )(p, nd, b2)


def kernel(x, edge_index, W1, b1, W2, b2):
    n, _ = x.shape
    hid = W1.shape[1]
    ncls = W2.shape[1]
    e = edge_index.shape[1]
    nchunk = -(-e // (NW * CH))
    nchunk = ((nchunk + 3) // 4) * 4
    e_pad = nchunk * NW * CH
    n_pad = ((n + 1 + 127) // 128) * 128

    src = edge_index[0]
    dst = edge_index[1]
    trash = jnp.full((e_pad - e,), n, jnp.int32)
    src3 = jnp.concatenate([src, trash]).reshape(NW, nchunk, CH)
    dst3 = jnp.concatenate([dst, trash]).reshape(NW, nchunk, CH)

    deg_s, deg_d = _degrees(src3, dst3, n_pad)
    ns, nd = _norms(deg_s, deg_d, n_pad)

    t1 = _scale_matmul(x, ns, W1, n)
    t1p = jnp.concatenate([t1, jnp.zeros((n_pad - n, hid), _f32)])
    rps = n_pad // NS
    p1 = _aggregate(t1p, src3, dst3, jnp.zeros((rps, hid), _f32), n)

    t2 = _mid(p1, nd, b1.reshape(1, hid), ns, W2, n)
    t2p = jnp.concatenate([t2, jnp.zeros((n_pad - n, ncls), _f32)])
    p2 = _aggregate(t2p, src3, dst3, jnp.zeros((rps, ncls), _f32), n)

    return _final(p2, nd, b2.reshape(1, ncls), n)


# speedup vs baseline: 1.2531x; 1.2531x over previous
import functools

import jax
import jax.numpy as jnp
from jax import lax
from jax.experimental import pallas as pl
from jax.experimental.pallas import tpu as pltpu
from jax.experimental.pallas import tpu_sc as plsc

NC = 2
NS = 16
NW = NC * NS
CH = 128

_f32 = jnp.float32

_SC_PARAMS = pltpu.CompilerParams(
    needs_layout_passes=False, use_tc_tiling_on_sc=False)


def _degrees(srcT, dstT, n_pad):
    ns_, nct, ch = srcT.shape
    nh = nct // NC
    mesh = plsc.VectorSubcoreMesh(core_axis_name="c", subcore_axis_name="s")

    @functools.partial(
        pl.kernel,
        out_type=(
            jax.ShapeDtypeStruct((NW, n_pad), _f32),
            jax.ShapeDtypeStruct((NW, n_pad), _f32),
        ),
        mesh=mesh,
        compiler_params=_SC_PARAMS,
        scratch_types=[
            pltpu.VMEM((nh, ch), jnp.int32),
            pltpu.VMEM((nh, ch), jnp.int32),
            pltpu.VMEM((n_pad,), _f32),
            pltpu.VMEM((n_pad,), _f32),
        ],
    )
    def deg_kernel(src_hbm, dst_hbm, out_s, out_d, src_v, dst_v, acc_s, acc_d):
        c = lax.axis_index("c")
        s = lax.axis_index("s")
        wid = c * NS + s
        pltpu.sync_copy(src_hbm.at[s, pl.ds(c * nh, nh)], src_v)
        pltpu.sync_copy(dst_hbm.at[s, pl.ds(c * nh, nh)], dst_v)
        zeros16 = jnp.zeros((16,), _f32)

        def zbody(i, carry):
            acc_s[pl.ds(i * 16, 16)] = zeros16
            acc_d[pl.ds(i * 16, 16)] = zeros16
            return carry

        lax.fori_loop(0, n_pad // 16, zbody, 0)
        ones16 = jnp.ones((16,), _f32)

        def cbody(i, carry):
            for j in range(ch // 16):
                si = src_v[i, pl.ds(j * 16, 16)]
                plsc.addupdate_scatter(acc_s, [si], ones16)
                di = dst_v[i, pl.ds(j * 16, 16)]
                plsc.addupdate_scatter(acc_d, [di], ones16)
            return carry

        lax.fori_loop(0, nh, cbody, 0)
        pltpu.sync_copy(acc_s, out_s.at[wid])
        pltpu.sync_copy(acc_d, out_d.at[wid])

    return deg_kernel(srcT, dstT)


def _aggregate(ta, tb, srcT, dstT, z_half, n_pad):
    npad_, dh = ta.shape
    ns_, nct, ch = srcT.shape
    nhalf = nct // 2
    rps = n_pad // NS
    mesh = plsc.VectorSubcoreMesh(core_axis_name="c", subcore_axis_name="s")

    @functools.partial(
        pl.kernel,
        out_type=jax.ShapeDtypeStruct((NC, n_pad, dh), _f32),
        mesh=mesh,
        compiler_params=_SC_PARAMS,
        scratch_types=[
            pltpu.VMEM((nhalf, ch), jnp.int32),
            pltpu.VMEM((nhalf, ch), jnp.int32),
            pltpu.VMEM((ch, dh), _f32),
            pltpu.VMEM_SHARED((n_pad, dh), _f32),
            pltpu.SemaphoreType.DMA,
        ],
    )
    def agg_kernel(ta_hbm, tb_hbm, src_hbm, dst_hbm, z_hbm, out_hbm,
                   src_v, dst_v, rows_v, sp_a, sem):
        c = lax.axis_index("c")
        s = lax.axis_index("s")
        row0 = s * rps
        pltpu.sync_copy(z_hbm, sp_a.at[pl.ds(row0, rps)])
        plsc.subcore_barrier()

        def run_half(h):
            pltpu.sync_copy(src_hbm.at[s, pl.ds(h * nhalf, nhalf)], src_v)
            pltpu.sync_copy(dst_hbm.at[s, pl.ds(h * nhalf, nhalf)], dst_v)

            def body(i, carry):
                @pl.when(c == 0)
                def _():
                    pltpu.async_copy(
                        ta_hbm.at[src_v.at[i]], rows_v, sem).wait()

                @pl.when(c == 1)
                def _():
                    pltpu.async_copy(
                        tb_hbm.at[src_v.at[i]], rows_v, sem).wait()

                pltpu.sync_copy(rows_v, sp_a.at[dst_v.at[i]], add=True)
                return carry

            lax.fori_loop(0, nhalf, body, 0)

        run_half(0)
        run_half(1)
        plsc.subcore_barrier()
        pltpu.sync_copy(sp_a.at[pl.ds(row0, rps)],
                        out_hbm.at[c, pl.ds(row0, rps)])

    return agg_kernel(ta, tb, srcT, dstT, z_half)


def _norms(deg_s, deg_d, n_pad):
    nw = deg_s.shape[0]

    def body(ds_ref, dd_ref, ns_ref, nd_ref):
        ones = jnp.ones((nw, 1), _f32)
        dn = (((0,), (0,)), ((), ()))
        ssum = lax.dot_general(ds_ref[...], ones, dn,
                               preferred_element_type=_f32)
        dsum = lax.dot_general(dd_ref[...], ones, dn,
                               preferred_element_type=_f32)
        ns_ref[...] = lax.rsqrt(jnp.maximum(ssum, 1.0))
        nd_ref[...] = lax.rsqrt(jnp.maximum(dsum, 1.0))

    return pl.pallas_call(
        body,
        out_shape=[
            jax.ShapeDtypeStruct((n_pad, 1), _f32),
            jax.ShapeDtypeStruct((n_pad, 1), _f32),
        ],
    )(deg_s, deg_d)


def _row_block(n):
    for b in (1264, 2000, 1000, 500, 250, 200, 100, 50, 25, 8):
        if n % b == 0:
            return b
    return n


def _scale_matmul(x, ns, wa, wb, n_pad):
    rb = _row_block(n_pad)
    d_in = x.shape[1]
    dh = wa.shape[1]

    def body(x_ref, ns_ref, wa_ref, wb_ref, oa_ref, ob_ref):
        xb = x_ref[...] * ns_ref[...]
        oa_ref[...] = jnp.dot(xb, wa_ref[...], preferred_element_type=_f32)
        ob_ref[...] = jnp.dot(xb, wb_ref[...], preferred_element_type=_f32)

    return pl.pallas_call(
        body,
        grid=(n_pad // rb,),
        in_specs=[
            pl.BlockSpec((rb, d_in), lambda i: (i, 0)),
            pl.BlockSpec((rb, 1), lambda i: (i, 0)),
            pl.BlockSpec((d_in, dh), lambda i: (0, 0)),
            pl.BlockSpec((d_in, dh), lambda i: (0, 0)),
        ],
        out_specs=[
            pl.BlockSpec((rb, dh), lambda i: (i, 0)),
            pl.BlockSpec((rb, dh), lambda i: (i, 0)),
        ],
        out_shape=[
            jax.ShapeDtypeStruct((n_pad, dh), _f32),
            jax.ShapeDtypeStruct((n_pad, dh), _f32),
        ],
    )(x, ns, wa, wb)


def _mid(p, nd, b1, ns, w2a, w2b, n_pad):
    rb = _row_block(n_pad)
    nc, _, dh = p.shape
    do = w2a.shape[1]

    def body(p_ref, nd_ref, b_ref, ns_ref, wa_ref, wb_ref, oa_ref, ob_ref):
        h = jnp.concatenate([p_ref[0], p_ref[1]], axis=1)
        h = jnp.maximum(h * nd_ref[...] + b_ref[...], 0.0)
        hs = h * ns_ref[...]
        oa_ref[...] = jnp.dot(hs, wa_ref[...], preferred_element_type=_f32)
        ob_ref[...] = jnp.dot(hs, wb_ref[...], preferred_element_type=_f32)

    return pl.pallas_call(
        body,
        grid=(n_pad // rb,),
        in_specs=[
            pl.BlockSpec((nc, rb, dh), lambda i: (0, i, 0)),
            pl.BlockSpec((rb, 1), lambda i: (i, 0)),
            pl.BlockSpec((1, 2 * dh), lambda i: (0, 0)),
            pl.BlockSpec((rb, 1), lambda i: (i, 0)),
            pl.BlockSpec((2 * dh, do), lambda i: (0, 0)),
            pl.BlockSpec((2 * dh, do), lambda i: (0, 0)),
        ],
        out_specs=[
            pl.BlockSpec((rb, do), lambda i: (i, 0)),
            pl.BlockSpec((rb, do), lambda i: (i, 0)),
        ],
        out_shape=[
            jax.ShapeDtypeStruct((n_pad, do), _f32),
            jax.ShapeDtypeStruct((n_pad, do), _f32),
        ],
    )(p, nd, b1, ns, w2a, w2b)


def _final(p, nd, b2, n):
    rb = _row_block(n)
    nc, _, dh = p.shape

    def body(p_ref, nd_ref, b_ref, o_ref):
        o = jnp.concatenate([p_ref[0], p_ref[1]], axis=1)
        o_ref[...] = o * nd_ref[...] + b_ref[...]

    return pl.pallas_call(
        body,
        grid=(n // rb,),
        in_specs=[
            pl.BlockSpec((nc, rb, dh), lambda i: (0, i, 0)),
            pl.BlockSpec((rb, 1), lambda i: (i, 0)),
            pl.BlockSpec((1, 2 * dh), lambda i: (0, 0)),
        ],
        out_specs=pl.BlockSpec((rb, 2 * dh), lambda i: (i, 0)),
        out_shape=jax.ShapeDtypeStruct((n, 2 * dh), _f32),
    )(p, nd, b2)


def kernel(x, edge_index, W1, b1, W2, b2):
    n, d_in = x.shape
    hid = W1.shape[1]
    ncls = W2.shape[1]
    e = edge_index.shape[1]
    nct = -(-e // (NS * CH))
    nct = ((nct + 1) // 2) * 2
    e_pad = nct * NS * CH
    n_pad = ((n + 1 + 127) // 128) * 128
    rps = n_pad // NS

    src = edge_index[0]
    dst = edge_index[1]
    trash = jnp.full((e_pad - e,), n, jnp.int32)
    srcT = jnp.concatenate([src, trash]).reshape(NS, nct, CH)
    dstT = jnp.concatenate([dst, trash]).reshape(NS, nct, CH)

    deg_s, deg_d = _degrees(srcT, dstT, n_pad)
    ns, nd = _norms(deg_s, deg_d, n_pad)

    x_pad = jnp.concatenate([x, jnp.zeros((n_pad - n, d_in), _f32)])
    h1 = hid // 2
    t1a, t1b = _scale_matmul(x_pad, ns, W1[:, :h1], W1[:, h1:], n_pad)
    p1 = _aggregate(t1a, t1b, srcT, dstT, jnp.zeros((rps, h1), _f32), n_pad)

    h2 = ((-(-ncls // 2)) + 7) // 8 * 8
    ncp = 2 * h2
    W2p = jnp.concatenate([W2, jnp.zeros((hid, ncp - ncls), _f32)], axis=1)
    b2p = jnp.concatenate([b2, jnp.zeros((ncp - ncls,), _f32)])
    t2a, t2b = _mid(p1, nd, b1.reshape(1, hid), ns,
                    W2p[:, :h2], W2p[:, h2:], n_pad)
    p2 = _aggregate(t2a, t2b, srcT, dstT, jnp.zeros((rps, h2), _f32), n_pad)

    out = _final(p2, nd, b2p.reshape(1, ncp), n)
    return out[:, :ncls]


# speedup vs baseline: 1.7901x; 1.4286x over previous
import functools

import jax
import jax.numpy as jnp
from jax import lax
from jax.experimental import pallas as pl
from jax.experimental.pallas import tpu as pltpu
from jax.experimental.pallas import tpu_sc as plsc

NC = 2
NS = 16
NW = NC * NS
CH = 128

_f32 = jnp.float32

_SC_PARAMS = pltpu.CompilerParams(
    needs_layout_passes=False, use_tc_tiling_on_sc=False)


def _degrees(srcT, dstT, n_pad):
    ns_, nct, ch = srcT.shape
    nh = nct // NC
    mesh = plsc.VectorSubcoreMesh(core_axis_name="c", subcore_axis_name="s")

    @functools.partial(
        pl.kernel,
        out_type=(
            jax.ShapeDtypeStruct((NW, n_pad), _f32),
            jax.ShapeDtypeStruct((NW, n_pad), _f32),
        ),
        mesh=mesh,
        compiler_params=_SC_PARAMS,
        scratch_types=[
            pltpu.VMEM((nh, ch), jnp.int32),
            pltpu.VMEM((nh, ch), jnp.int32),
            pltpu.VMEM((n_pad,), _f32),
            pltpu.VMEM((n_pad,), _f32),
        ],
    )
    def deg_kernel(src_hbm, dst_hbm, out_s, out_d, src_v, dst_v, acc_s, acc_d):
        c = lax.axis_index("c")
        s = lax.axis_index("s")
        wid = c * NS + s
        pltpu.sync_copy(src_hbm.at[s, pl.ds(c * nh, nh)], src_v)
        pltpu.sync_copy(dst_hbm.at[s, pl.ds(c * nh, nh)], dst_v)
        zeros16 = jnp.zeros((16,), _f32)

        def zbody(i, carry):
            acc_s[pl.ds(i * 16, 16)] = zeros16
            acc_d[pl.ds(i * 16, 16)] = zeros16
            return carry

        lax.fori_loop(0, n_pad // 16, zbody, 0)
        ones16 = jnp.ones((16,), _f32)

        def cbody(i, carry):
            for j in range(ch // 16):
                si = src_v[i, pl.ds(j * 16, 16)]
                plsc.addupdate_scatter(acc_s, [si], ones16)
                di = dst_v[i, pl.ds(j * 16, 16)]
                plsc.addupdate_scatter(acc_d, [di], ones16)
            return carry

        lax.fori_loop(0, nh, cbody, 0)
        pltpu.sync_copy(acc_s, out_s.at[wid])
        pltpu.sync_copy(acc_d, out_d.at[wid])

    return deg_kernel(srcT, dstT)


def _aggregate(ta, tb, srcT, dstT, z_half, n_pad):
    npad_, dh = ta.shape
    ns_, nct, ch = srcT.shape
    nhalf = nct // 2
    rps = n_pad // NS
    mesh = plsc.VectorSubcoreMesh(core_axis_name="c", subcore_axis_name="s")

    @functools.partial(
        pl.kernel,
        out_type=jax.ShapeDtypeStruct((NC, n_pad, dh), _f32),
        mesh=mesh,
        compiler_params=_SC_PARAMS,
        scratch_types=[
            pltpu.VMEM((nhalf, ch), jnp.int32),
            pltpu.VMEM((nhalf, ch), jnp.int32),
            pltpu.VMEM((ch, dh), _f32),
            pltpu.VMEM_SHARED((n_pad, dh), _f32),
            pltpu.VMEM_SHARED((n_pad, dh), _f32),
            pltpu.SemaphoreType.DMA,
        ],
    )
    def agg_kernel(ta_hbm, tb_hbm, src_hbm, dst_hbm, z_hbm, out_hbm,
                   src_v, dst_v, rows_v, sp_t, sp_a, sem):
        c = lax.axis_index("c")
        s = lax.axis_index("s")
        row0 = s * rps
        @pl.when(c == 0)
        def _():
            pltpu.sync_copy(ta_hbm.at[pl.ds(row0, rps)],
                            sp_t.at[pl.ds(row0, rps)])

        @pl.when(c == 1)
        def _():
            pltpu.sync_copy(tb_hbm.at[pl.ds(row0, rps)],
                            sp_t.at[pl.ds(row0, rps)])

        pltpu.sync_copy(z_hbm, sp_a.at[pl.ds(row0, rps)])
        plsc.subcore_barrier()

        def run_half(h):
            pltpu.sync_copy(src_hbm.at[s, pl.ds(h * nhalf, nhalf)], src_v)
            pltpu.sync_copy(dst_hbm.at[s, pl.ds(h * nhalf, nhalf)], dst_v)

            def body(i, carry):
                pltpu.async_copy(sp_t.at[src_v.at[i]], rows_v, sem).wait()
                pltpu.sync_copy(rows_v, sp_a.at[dst_v.at[i]], add=True)
                return carry

            lax.fori_loop(0, nhalf, body, 0)

        run_half(0)
        run_half(1)
        plsc.subcore_barrier()
        pltpu.sync_copy(sp_a.at[pl.ds(row0, rps)],
                        out_hbm.at[c, pl.ds(row0, rps)])

    return agg_kernel(ta, tb, srcT, dstT, z_half)


def _norms(deg_s, deg_d, n_pad):
    nw = deg_s.shape[0]

    def body(ds_ref, dd_ref, ns_ref, nd_ref):
        ones = jnp.ones((nw, 1), _f32)
        dn = (((0,), (0,)), ((), ()))
        ssum = lax.dot_general(ds_ref[...], ones, dn,
                               preferred_element_type=_f32)
        dsum = lax.dot_general(dd_ref[...], ones, dn,
                               preferred_element_type=_f32)
        ns_ref[...] = lax.rsqrt(jnp.maximum(ssum, 1.0))
        nd_ref[...] = lax.rsqrt(jnp.maximum(dsum, 1.0))

    return pl.pallas_call(
        body,
        out_shape=[
            jax.ShapeDtypeStruct((n_pad, 1), _f32),
            jax.ShapeDtypeStruct((n_pad, 1), _f32),
        ],
    )(deg_s, deg_d)


def _row_block(n):
    for b in (1264, 2000, 1000, 500, 250, 200, 100, 50, 25, 8):
        if n % b == 0:
            return b
    return n


def _scale_matmul(x, ns, wa, wb, n_pad):
    rb = _row_block(n_pad)
    d_in = x.shape[1]
    dh = wa.shape[1]

    def body(x_ref, ns_ref, wa_ref, wb_ref, oa_ref, ob_ref):
        xb = x_ref[...] * ns_ref[...]
        oa_ref[...] = jnp.dot(xb, wa_ref[...], preferred_element_type=_f32)
        ob_ref[...] = jnp.dot(xb, wb_ref[...], preferred_element_type=_f32)

    return pl.pallas_call(
        body,
        grid=(n_pad // rb,),
        in_specs=[
            pl.BlockSpec((rb, d_in), lambda i: (i, 0)),
            pl.BlockSpec((rb, 1), lambda i: (i, 0)),
            pl.BlockSpec((d_in, dh), lambda i: (0, 0)),
            pl.BlockSpec((d_in, dh), lambda i: (0, 0)),
        ],
        out_specs=[
            pl.BlockSpec((rb, dh), lambda i: (i, 0)),
            pl.BlockSpec((rb, dh), lambda i: (i, 0)),
        ],
        out_shape=[
            jax.ShapeDtypeStruct((n_pad, dh), _f32),
            jax.ShapeDtypeStruct((n_pad, dh), _f32),
        ],
    )(x, ns, wa, wb)


def _mid(p, nd, b1, ns, w2a, w2b, n_pad):
    rb = _row_block(n_pad)
    nc, _, dh = p.shape
    do = w2a.shape[1]

    def body(p_ref, nd_ref, b_ref, ns_ref, wa_ref, wb_ref, oa_ref, ob_ref):
        h = jnp.concatenate([p_ref[0], p_ref[1]], axis=1)
        h = jnp.maximum(h * nd_ref[...] + b_ref[...], 0.0)
        hs = h * ns_ref[...]
        oa_ref[...] = jnp.dot(hs, wa_ref[...], preferred_element_type=_f32)
        ob_ref[...] = jnp.dot(hs, wb_ref[...], preferred_element_type=_f32)

    return pl.pallas_call(
        body,
        grid=(n_pad // rb,),
        in_specs=[
            pl.BlockSpec((nc, rb, dh), lambda i: (0, i, 0)),
            pl.BlockSpec((rb, 1), lambda i: (i, 0)),
            pl.BlockSpec((1, 2 * dh), lambda i: (0, 0)),
            pl.BlockSpec((rb, 1), lambda i: (i, 0)),
            pl.BlockSpec((2 * dh, do), lambda i: (0, 0)),
            pl.BlockSpec((2 * dh, do), lambda i: (0, 0)),
        ],
        out_specs=[
            pl.BlockSpec((rb, do), lambda i: (i, 0)),
            pl.BlockSpec((rb, do), lambda i: (i, 0)),
        ],
        out_shape=[
            jax.ShapeDtypeStruct((n_pad, do), _f32),
            jax.ShapeDtypeStruct((n_pad, do), _f32),
        ],
    )(p, nd, b1, ns, w2a, w2b)


def _final(p, nd, b2, n):
    rb = _row_block(n)
    nc, _, dh = p.shape

    def body(p_ref, nd_ref, b_ref, o_ref):
        o = jnp.concatenate([p_ref[0], p_ref[1]], axis=1)
        o_ref[...] = o * nd_ref[...] + b_ref[...]

    return pl.pallas_call(
        body,
        grid=(n // rb,),
        in_specs=[
            pl.BlockSpec((nc, rb, dh), lambda i: (0, i, 0)),
            pl.BlockSpec((rb, 1), lambda i: (i, 0)),
            pl.BlockSpec((1, 2 * dh), lambda i: (0, 0)),
        ],
        out_specs=pl.BlockSpec((rb, 2 * dh), lambda i: (i, 0)),
        out_shape=jax.ShapeDtypeStruct((n, 2 * dh), _f32),
    )(p, nd, b2)


def kernel(x, edge_index, W1, b1, W2, b2):
    n, d_in = x.shape
    hid = W1.shape[1]
    ncls = W2.shape[1]
    e = edge_index.shape[1]
    nct = -(-e // (NS * CH))
    nct = ((nct + 1) // 2) * 2
    e_pad = nct * NS * CH
    n_pad = ((n + 1 + 127) // 128) * 128
    rps = n_pad // NS

    src = edge_index[0]
    dst = edge_index[1]
    trash = jnp.full((e_pad - e,), n, jnp.int32)
    srcT = jnp.concatenate([src, trash]).reshape(NS, nct, CH)
    dstT = jnp.concatenate([dst, trash]).reshape(NS, nct, CH)

    deg_s, deg_d = _degrees(srcT, dstT, n_pad)
    ns, nd = _norms(deg_s, deg_d, n_pad)

    x_pad = jnp.concatenate([x, jnp.zeros((n_pad - n, d_in), _f32)])
    h1 = hid // 2
    t1a, t1b = _scale_matmul(x_pad, ns, W1[:, :h1], W1[:, h1:], n_pad)
    p1 = _aggregate(t1a, t1b, srcT, dstT, jnp.zeros((rps, h1), _f32), n_pad)

    h2 = ((-(-ncls // 2)) + 7) // 8 * 8
    ncp = 2 * h2
    W2p = jnp.concatenate([W2, jnp.zeros((hid, ncp - ncls), _f32)], axis=1)
    b2p = jnp.concatenate([b2, jnp.zeros((ncp - ncls,), _f32)])
    t2a, t2b = _mid(p1, nd, b1.reshape(1, hid), ns,
                    W2p[:, :h2], W2p[:, h2:], n_pad)
    p2 = _aggregate(t2a, t2b, srcT, dstT, jnp.zeros((rps, h2), _f32), n_pad)

    out = _final(p2, nd, b2p.reshape(1, ncp), n)
    return out[:, :ncls]
